# baseline (speedup 1.0000x reference)
import functools

import jax
import jax.numpy as jnp
from jax import lax
from jax.experimental import pallas as pl
from jax.experimental.pallas import tpu as pltpu
from jax.experimental.pallas import tpu_sc as plsc

D = 64
BATCH = 16384
NC, NS, L = 2, 16, 16
NW = NC * NS
BPW = BATCH // NW

_mesh = plsc.VectorSubcoreMesh(core_axis_name="c", subcore_axis_name="s")


@functools.partial(
    pl.kernel,
    mesh=_mesh,
    out_type=jax.ShapeDtypeStruct((BATCH, D), jnp.float32),
    scratch_types=[
        pltpu.VMEM((BPW,), jnp.int32),
        pltpu.VMEM((BPW, D), jnp.float32),
        pltpu.VMEM((BPW, D), jnp.float32),
        pltpu.SemaphoreType.DMA,
    ],
    compiler_params=pltpu.CompilerParams(use_tc_tiling_on_sc=False),
)
def _proto_add(in_hbm, idx_hbm, table_hbm, out_hbm, idx_v, acc_v, rows_v, sem):
    wid = lax.axis_index("s") * NC + lax.axis_index("c")
    base = wid * BPW
    pltpu.sync_copy(idx_hbm.at[pl.ds(base, BPW)], idx_v)
    gather = pltpu.async_copy(table_hbm.at[idx_v], rows_v, sem)
    pltpu.sync_copy(in_hbm.at[pl.ds(base, BPW)], acc_v)
    gather.wait()

    def body(i, carry):
        for j in range(D // L):
            s = pl.ds(j * L, L)
            acc_v[i, s] = acc_v[i, s] + rows_v[i, s]
        return carry

    lax.fori_loop(0, BPW, body, 0)
    pltpu.sync_copy(acc_v, out_hbm.at[pl.ds(base, BPW)])


def kernel(in_repr, mask_idx, prototype_knobs):
    return _proto_add(in_repr, mask_idx.astype(jnp.int32), prototype_knobs)


# speedup vs baseline: 2.1369x; 2.1369x over previous
import functools

import jax
import jax.numpy as jnp
from jax import lax
from jax.experimental import pallas as pl
from jax.experimental.pallas import tpu as pltpu
from jax.experimental.pallas import tpu_sc as plsc

D = 64
BATCH = 16384
NROWS = 100000
NC, NS, L = 2, 16, 16
NW = NC * NS
PASSES = D // NW
CHUNK = 8192
UNROLL = 8

_mesh = plsc.VectorSubcoreMesh(core_axis_name="c", subcore_axis_name="s")


@functools.partial(
    pl.kernel,
    mesh=_mesh,
    out_type=jax.ShapeDtypeStruct((D, BATCH), jnp.float32),
    scratch_types=[
        pltpu.VMEM((NROWS,), jnp.float32),
        pltpu.VMEM((BATCH,), jnp.int32),
        pltpu.VMEM((CHUNK,), jnp.float32),
        pltpu.SemaphoreType.DMA,
    ],
    compiler_params=pltpu.CompilerParams(
        use_tc_tiling_on_sc=True, needs_layout_passes=False
    ),
)
def _proto_add_t(in_t, idx_hbm, knobs_t, out_t, row_v, idx_v, io_v, sem):
    w = lax.axis_index("s") * NC + lax.axis_index("c")
    pltpu.sync_copy(idx_hbm, idx_v)

    def one_pass(p, carry):
        c = p * NW + w
        pltpu.sync_copy(knobs_t.at[c], row_v)

        for b0 in range(0, BATCH, CHUNK):
            pltpu.sync_copy(in_t.at[c, pl.ds(b0, CHUNK)], io_v)

            def body(k, carry3, b0=b0):
                base = k * (L * UNROLL)
                for u in range(UNROLL):
                    s = pl.ds(base + u * L, L)
                    i16 = idx_v[pl.ds(b0 + base + u * L, L)]
                    g = plsc.load_gather(row_v, [i16])
                    io_v[s] = io_v[s] + g
                return carry3

            lax.fori_loop(0, CHUNK // (L * UNROLL), body, 0)
            pltpu.sync_copy(io_v, out_t.at[c, pl.ds(b0, CHUNK)])
        return carry

    lax.fori_loop(0, PASSES, one_pass, 0)


def kernel(in_repr, mask_idx, prototype_knobs):
    out_t = _proto_add_t(in_repr.T, mask_idx.astype(jnp.int32), prototype_knobs.T)
    return out_t.T


# speedup vs baseline: 2.2272x; 1.0422x over previous
import functools

import jax
import jax.numpy as jnp
from jax import lax
from jax.experimental import pallas as pl
from jax.experimental.pallas import tpu as pltpu
from jax.experimental.pallas import tpu_sc as plsc

D = 64
BATCH = 16384
NROWS = 100000
NC, NS, L = 2, 16, 16
NW = NC * NS
PASSES = D // NW
CHUNK = 2048
NCH = BATCH // CHUNK
UNROLL = 8

_mesh = plsc.VectorSubcoreMesh(core_axis_name="c", subcore_axis_name="s")


@functools.partial(
    pl.kernel,
    mesh=_mesh,
    out_type=jax.ShapeDtypeStruct((D, BATCH), jnp.float32),
    scratch_types=[
        pltpu.VMEM((NROWS,), jnp.float32),
        pltpu.VMEM((BATCH,), jnp.int32),
        pltpu.VMEM((CHUNK,), jnp.float32),
        pltpu.VMEM((CHUNK,), jnp.float32),
        pltpu.VMEM((CHUNK,), jnp.float32),
        pltpu.VMEM((CHUNK,), jnp.float32),
        pltpu.SemaphoreType.DMA,
        pltpu.SemaphoreType.DMA,
        pltpu.SemaphoreType.DMA,
        pltpu.SemaphoreType.DMA,
        pltpu.SemaphoreType.DMA,
        pltpu.SemaphoreType.DMA,
    ],
    compiler_params=pltpu.CompilerParams(
        use_tc_tiling_on_sc=True,
        needs_layout_passes=False,
        disable_bounds_checks=True,
    ),
)
def _proto_add_t(in_t, idx_hbm, knobs_t, out_t,
                 row_v, idx_v, ib0, ib1, ob0, ob1,
                 sem_row, sem_idx, sem_i0, sem_i1, sem_o0, sem_o1):
    w = lax.axis_index("s") * NC + lax.axis_index("c")
    ib = (ib0, ib1)
    ob = (ob0, ob1)
    sem_i = (sem_i0, sem_i1)
    sem_o = (sem_o0, sem_o1)

    def feat(p):
        return p * NW + w

    def in_copy(p, i):
        return pltpu.async_copy(
            in_t.at[feat(p), pl.ds(i * CHUNK, CHUNK)], ib[i % 2], sem_i[i % 2]
        )

    h_row = pltpu.async_copy(knobs_t.at[feat(0)], row_v, sem_row)
    h_idx = pltpu.async_copy(idx_hbm, idx_v, sem_idx)
    h_in = [in_copy(0, 0), in_copy(0, 1)]
    h_out = [None, None]

    h_idx.wait()
    for p in range(PASSES):
        h_row.wait()
        for i in range(NCH):
            h_in[i % 2].wait()
            if h_out[i % 2] is not None:
                h_out[i % 2].wait()

            def body(k, carry, b0=i * CHUNK, ibuf=ib[i % 2], obuf=ob[i % 2]):
                base = k * (L * UNROLL)
                for u in range(UNROLL):
                    s = pl.ds(base + u * L, L)
                    i16 = idx_v[pl.ds(b0 + base + u * L, L)]
                    g = plsc.load_gather(row_v, [i16])
                    obuf[s] = ibuf[s] + g
                return carry

            lax.fori_loop(0, CHUNK // (L * UNROLL), body, 0)
            h_out[i % 2] = pltpu.async_copy(
                ob[i % 2], out_t.at[feat(p), pl.ds(i * CHUNK, CHUNK)], sem_o[i % 2]
            )
            if p == PASSES - 1 and i == NCH - 1:
                pass
            elif i == NCH - 1:
                h_row = pltpu.async_copy(knobs_t.at[feat(p + 1)], row_v, sem_row)
            nxt = p * NCH + i + 2
            if nxt < PASSES * NCH:
                h_in[i % 2] = in_copy(nxt // NCH, nxt % NCH)
    h_out[0].wait()
    h_out[1].wait()


def kernel(in_repr, mask_idx, prototype_knobs):
    out_t = _proto_add_t(in_repr.T, mask_idx.astype(jnp.int32), prototype_knobs.T)
    return out_t.T


# speedup vs baseline: 2.2372x; 1.0045x over previous
import functools

import jax
import jax.numpy as jnp
from jax import lax
from jax.experimental import pallas as pl
from jax.experimental.pallas import tpu as pltpu
from jax.experimental.pallas import tpu_sc as plsc

D = 64
BATCH = 16384
NROWS = 100000
NC, NS, L = 2, 16, 16
NW = NC * NS
PASSES = D // NW
CHUNK = 2048
NCH = BATCH // CHUNK
UNROLL = 8

_mesh = plsc.VectorSubcoreMesh(core_axis_name="c", subcore_axis_name="s")


@functools.partial(
    pl.kernel,
    mesh=_mesh,
    out_type=jax.ShapeDtypeStruct((D, BATCH), jnp.float32),
    scratch_types=[
        pltpu.VMEM((NROWS,), jnp.float32),
        pltpu.VMEM((BATCH,), jnp.int32),
        pltpu.VMEM((CHUNK,), jnp.float32),
        pltpu.VMEM((CHUNK,), jnp.float32),
        pltpu.VMEM((CHUNK,), jnp.float32),
        pltpu.VMEM((CHUNK,), jnp.float32),
        pltpu.SemaphoreType.DMA,
        pltpu.SemaphoreType.DMA,
        pltpu.SemaphoreType.DMA,
        pltpu.SemaphoreType.DMA,
        pltpu.SemaphoreType.DMA,
        pltpu.SemaphoreType.DMA,
    ],
    compiler_params=pltpu.CompilerParams(
        use_tc_tiling_on_sc=True,
        needs_layout_passes=False,
        disable_bounds_checks=True,
        disable_semaphore_checks=True,
        skip_device_barrier=True,
    ),
)
def _proto_add_t(in_t, idx_hbm, knobs_t, out_t,
                 row_v, idx_v, ib0, ib1, ob0, ob1,
                 sem_row, sem_idx, sem_i0, sem_i1, sem_o0, sem_o1):
    w = lax.axis_index("s") * NC + lax.axis_index("c")
    ib = (ib0, ib1)
    ob = (ob0, ob1)
    sem_i = (sem_i0, sem_i1)
    sem_o = (sem_o0, sem_o1)

    def feat(p):
        return p * NW + w

    def in_copy(p, i):
        return pltpu.async_copy(
            in_t.at[feat(p), pl.ds(i * CHUNK, CHUNK)], ib[i % 2], sem_i[i % 2]
        )

    h_row = pltpu.async_copy(knobs_t.at[feat(0)], row_v, sem_row)
    h_idx = pltpu.async_copy(idx_hbm, idx_v, sem_idx)
    h_in = [in_copy(0, 0), in_copy(0, 1)]
    h_out = [None, None]

    h_idx.wait()
    for p in range(PASSES):
        h_row.wait()
        for i in range(NCH):
            h_in[i % 2].wait()
            if h_out[i % 2] is not None:
                h_out[i % 2].wait()

            def body(k, carry, b0=i * CHUNK, ibuf=ib[i % 2], obuf=ob[i % 2]):
                base = k * (L * UNROLL)
                for u in range(UNROLL):
                    s = pl.ds(base + u * L, L)
                    i16 = idx_v[pl.ds(b0 + base + u * L, L)]
                    g = plsc.load_gather(row_v, [i16])
                    obuf[s] = ibuf[s] + g
                return carry

            lax.fori_loop(0, CHUNK // (L * UNROLL), body, 0)
            h_out[i % 2] = pltpu.async_copy(
                ob[i % 2], out_t.at[feat(p), pl.ds(i * CHUNK, CHUNK)], sem_o[i % 2]
            )
            if p == PASSES - 1 and i == NCH - 1:
                pass
            elif i == NCH - 1:
                h_row = pltpu.async_copy(knobs_t.at[feat(p + 1)], row_v, sem_row)
            nxt = p * NCH + i + 2
            if nxt < PASSES * NCH:
                h_in[i % 2] = in_copy(nxt // NCH, nxt % NCH)
    h_out[0].wait()
    h_out[1].wait()


def kernel(in_repr, mask_idx, prototype_knobs):
    out_t = _proto_add_t(in_repr.T, mask_idx.astype(jnp.int32), prototype_knobs.T)
    return out_t.T


# speedup vs baseline: 2.5681x; 1.1479x over previous
import functools

import jax
import jax.numpy as jnp
from jax import lax
from jax.experimental import pallas as pl
from jax.experimental.pallas import tpu as pltpu
from jax.experimental.pallas import tpu_sc as plsc

D = 64
BATCH = 16384
NROWS = 100000
NC, NS, L = 2, 16, 16
NW = NC * NS
PASSES = D // NW
CHUNK = 2048
NCH = BATCH // CHUNK
UNROLL = 8

_mesh = plsc.VectorSubcoreMesh(core_axis_name="c", subcore_axis_name="s")


@functools.partial(
    pl.kernel,
    mesh=_mesh,
    out_type=jax.ShapeDtypeStruct((D, BATCH), jnp.float32),
    scratch_types=[
        pltpu.VMEM((NROWS,), jnp.float32),
        pltpu.VMEM((BATCH,), jnp.int32),
        pltpu.VMEM((CHUNK,), jnp.float32),
        pltpu.VMEM((CHUNK,), jnp.float32),
        pltpu.VMEM((CHUNK,), jnp.float32),
        pltpu.VMEM((CHUNK,), jnp.float32),
        pltpu.SemaphoreType.DMA,
        pltpu.SemaphoreType.DMA,
        pltpu.SemaphoreType.DMA,
        pltpu.SemaphoreType.DMA,
        pltpu.SemaphoreType.DMA,
        pltpu.SemaphoreType.DMA,
    ],
    compiler_params=pltpu.CompilerParams(
        use_tc_tiling_on_sc=True,
        needs_layout_passes=False,
        disable_bounds_checks=True,
        disable_semaphore_checks=True,
        skip_device_barrier=True,
    ),
)
def _proto_add_t(in_t, idx_hbm, knobs_t, out_t,
                 row_v, idx_v, ib0, ib1, ob0, ob1,
                 sem_row, sem_idx, sem_i0, sem_i1, sem_o0, sem_o1):
    w = lax.axis_index("s") * NC + lax.axis_index("c")
    ib = (ib0, ib1)
    ob = (ob0, ob1)
    sem_i = (sem_i0, sem_i1)
    sem_o = (sem_o0, sem_o1)

    def feat(p):
        return p * NW + w

    def in_copy(p, i):
        return pltpu.async_copy(
            in_t.at[feat(p), pl.ds(i * CHUNK, CHUNK)], ib[i % 2], sem_i[i % 2]
        )

    h_row = pltpu.async_copy(knobs_t.at[feat(0)], row_v, sem_row)
    h_idx = pltpu.async_copy(idx_hbm, idx_v, sem_idx)
    h_in = [in_copy(0, 0), in_copy(0, 1)]
    h_out = [None, None]

    h_idx.wait()
    for p in range(PASSES):
        h_row.wait()
        for i in range(NCH):
            h_in[i % 2].wait()
            if h_out[i % 2] is not None:
                h_out[i % 2].wait()

            @plsc.parallel_loop(0, CHUNK, L, unroll=UNROLL)
            def _gather_add(k, b0=i * CHUNK, ibuf=ib[i % 2], obuf=ob[i % 2]):
                s = pl.ds(k, L)
                i16 = idx_v[pl.ds(b0 + k, L)]
                g = plsc.load_gather(row_v, [i16])
                obuf[s] = ibuf[s] + g
            h_out[i % 2] = pltpu.async_copy(
                ob[i % 2], out_t.at[feat(p), pl.ds(i * CHUNK, CHUNK)], sem_o[i % 2]
            )
            if p == PASSES - 1 and i == NCH - 1:
                pass
            elif i == NCH - 1:
                h_row = pltpu.async_copy(knobs_t.at[feat(p + 1)], row_v, sem_row)
            nxt = p * NCH + i + 2
            if nxt < PASSES * NCH:
                h_in[i % 2] = in_copy(nxt // NCH, nxt % NCH)
    h_out[0].wait()
    h_out[1].wait()


def kernel(in_repr, mask_idx, prototype_knobs):
    out_t = _proto_add_t(in_repr.T, mask_idx.astype(jnp.int32), prototype_knobs.T)
    return out_t.T


# speedup vs baseline: 2.6755x; 1.0418x over previous
import functools

import jax
import jax.numpy as jnp
from jax import lax
from jax.experimental import pallas as pl
from jax.experimental.pallas import tpu as pltpu
from jax.experimental.pallas import tpu_sc as plsc

D = 64
BATCH = 16384
NROWS = 100000
NC, NS, L = 2, 16, 16
NW = NC * NS
PASSES = D // NW
CHUNK = 2048
NCH = BATCH // CHUNK
UNROLL = 8

_mesh = plsc.VectorSubcoreMesh(core_axis_name="c", subcore_axis_name="s")


@functools.partial(
    pl.kernel,
    mesh=_mesh,
    out_type=jax.ShapeDtypeStruct((D, BATCH), jnp.float32),
    scratch_types=[
        pltpu.VMEM((NROWS,), jnp.float32),
        pltpu.VMEM((BATCH,), jnp.int32),
        pltpu.VMEM((CHUNK,), jnp.float32),
        pltpu.VMEM((CHUNK,), jnp.float32),
        pltpu.VMEM((CHUNK,), jnp.float32),
        pltpu.VMEM((CHUNK,), jnp.float32),
        pltpu.SemaphoreType.DMA,
        pltpu.SemaphoreType.DMA,
        pltpu.SemaphoreType.DMA,
        pltpu.SemaphoreType.DMA,
        pltpu.SemaphoreType.DMA,
        pltpu.SemaphoreType.DMA,
    ],
    compiler_params=pltpu.CompilerParams(
        use_tc_tiling_on_sc=True,
        needs_layout_passes=False,
        disable_bounds_checks=True,
        disable_semaphore_checks=True,
        skip_device_barrier=True,
    ),
)
def _proto_add_t(in_t, idx_hbm, knobs_t, out_t,
                 row_v, idx_v, ib0, ib1, ob0, ob1,
                 sem_row, sem_idx, sem_i0, sem_i1, sem_o0, sem_o1):
    w = lax.axis_index("s") * NC + lax.axis_index("c")
    ib = (ib0, ib1)
    ob = (ob0, ob1)
    sem_i = (sem_i0, sem_i1)
    sem_o = (sem_o0, sem_o1)
    c0 = w
    c1 = NW + w

    def in_copy(c, b0, par):
        return pltpu.async_copy(
            in_t.at[c].at[pl.ds(b0, CHUNK)], ib[par], sem_i[par]
        )

    def out_copy(c, b0, par):
        return pltpu.async_copy(
            ob[par], out_t.at[c].at[pl.ds(b0, CHUNK)], sem_o[par]
        )

    def in_wait(par):
        pltpu.make_async_copy(
            in_t.at[c0].at[pl.ds(0, CHUNK)], ib[par], sem_i[par]
        ).wait()

    def out_wait(par):
        pltpu.make_async_copy(
            ob[par], out_t.at[c0].at[pl.ds(0, CHUNK)], sem_o[par]
        ).wait()

    def row_wait():
        pltpu.make_async_copy(knobs_t.at[c0], row_v, sem_row).wait()

    def gather_chunk(b0, par):
        @plsc.parallel_loop(0, CHUNK, L, unroll=UNROLL)
        def _gather_add(k, ibuf=ib[par], obuf=ob[par]):
            s = pl.ds(k, L)
            i16 = idx_v[pl.ds(b0 + k, L)]
            g = plsc.load_gather(row_v, [i16])
            obuf[s] = ibuf[s] + g

    h_row = pltpu.async_copy(knobs_t.at[c0], row_v, sem_row)
    h_idx = pltpu.async_copy(idx_hbm, idx_v, sem_idx)
    in_copy(c0, 0, 0)
    in_copy(c0, CHUNK, 1)
    h_idx.wait()

    for p in range(PASSES):
        c = (c0, c1)[p]
        row_wait()

        def pair(j, carry, p=p, c=c):
            base = pl.multiple_of(j * (2 * CHUNK), 2 * CHUNK)
            for par in range(2):
                b0 = base + par * CHUNK
                in_wait(par)
                if p == 0:
                    @pl.when(j >= 1)
                    def _():
                        out_wait(par)
                else:
                    out_wait(par)
                gather_chunk(b0, par)
                out_copy(c, b0, par)
                nxt = b0 + 2 * CHUNK
                if p == 0:
                    nc = jnp.where(nxt >= BATCH, c1, c)
                    in_copy(nc, pl.multiple_of(lax.rem(nxt, BATCH), CHUNK), par)
                else:
                    @pl.when(nxt < BATCH)
                    def _():
                        in_copy(c, nxt, par)
            return carry

        lax.fori_loop(0, NCH // 2, pair, 0)
        if p == 0:
            h_row = pltpu.async_copy(knobs_t.at[c1], row_v, sem_row)
    out_wait(0)
    out_wait(1)


def kernel(in_repr, mask_idx, prototype_knobs):
    out_t = _proto_add_t(in_repr.T, mask_idx.astype(jnp.int32), prototype_knobs.T)
    return out_t.T


# speedup vs baseline: 2.6790x; 1.0013x over previous
import functools

import jax
import jax.numpy as jnp
from jax import lax
from jax.experimental import pallas as pl
from jax.experimental.pallas import tpu as pltpu
from jax.experimental.pallas import tpu_sc as plsc

D = 64
BATCH = 16384
NROWS = 100000
NC, NS, L = 2, 16, 16
NW = NC * NS
PASSES = D // NW
CHUNK = 2048
NCH = BATCH // CHUNK
UNROLL = 4

_mesh = plsc.VectorSubcoreMesh(core_axis_name="c", subcore_axis_name="s")


@functools.partial(
    pl.kernel,
    mesh=_mesh,
    out_type=jax.ShapeDtypeStruct((D, BATCH), jnp.float32),
    scratch_types=[
        pltpu.VMEM((NROWS,), jnp.float32),
        pltpu.VMEM((BATCH,), jnp.int32),
        pltpu.VMEM((CHUNK,), jnp.float32),
        pltpu.VMEM((CHUNK,), jnp.float32),
        pltpu.VMEM((CHUNK,), jnp.float32),
        pltpu.VMEM((CHUNK,), jnp.float32),
        pltpu.SemaphoreType.DMA,
        pltpu.SemaphoreType.DMA,
        pltpu.SemaphoreType.DMA,
        pltpu.SemaphoreType.DMA,
        pltpu.SemaphoreType.DMA,
        pltpu.SemaphoreType.DMA,
    ],
    compiler_params=pltpu.CompilerParams(
        use_tc_tiling_on_sc=True,
        needs_layout_passes=False,
        disable_bounds_checks=True,
        disable_semaphore_checks=True,
        skip_device_barrier=True,
    ),
)
def _proto_add_t(in_t, idx_hbm, knobs_t, out_t,
                 row_v, idx_v, ib0, ib1, ob0, ob1,
                 sem_row, sem_idx, sem_i0, sem_i1, sem_o0, sem_o1):
    w = lax.axis_index("s") * NC + lax.axis_index("c")
    ib = (ib0, ib1)
    ob = (ob0, ob1)
    sem_i = (sem_i0, sem_i1)
    sem_o = (sem_o0, sem_o1)
    c0 = w
    c1 = NW + w

    def in_copy(c, b0, par):
        return pltpu.async_copy(
            in_t.at[c].at[pl.ds(b0, CHUNK)], ib[par], sem_i[par]
        )

    def out_copy(c, b0, par):
        return pltpu.async_copy(
            ob[par], out_t.at[c].at[pl.ds(b0, CHUNK)], sem_o[par]
        )

    def in_wait(par):
        pltpu.make_async_copy(
            in_t.at[c0].at[pl.ds(0, CHUNK)], ib[par], sem_i[par]
        ).wait()

    def out_wait(par):
        pltpu.make_async_copy(
            ob[par], out_t.at[c0].at[pl.ds(0, CHUNK)], sem_o[par]
        ).wait()

    def row_wait():
        pltpu.make_async_copy(knobs_t.at[c0], row_v, sem_row).wait()

    def gather_chunk(b0, par):
        @plsc.parallel_loop(0, CHUNK, L, unroll=UNROLL)
        def _gather_add(k, ibuf=ib[par], obuf=ob[par]):
            s = pl.ds(k, L)
            i16 = idx_v[pl.ds(b0 + k, L)]
            g = plsc.load_gather(row_v, [i16])
            obuf[s] = ibuf[s] + g

    h_idx = pltpu.async_copy(idx_hbm, idx_v, sem_idx)
    in_copy(c0, 0, 0)
    in_copy(c0, CHUNK, 1)
    h_row = pltpu.async_copy(knobs_t.at[c0], row_v, sem_row)
    h_idx.wait()

    for p in range(PASSES):
        c = (c0, c1)[p]
        row_wait()

        def pair(j, carry, p=p, c=c):
            base = pl.multiple_of(j * (2 * CHUNK), 2 * CHUNK)
            for par in range(2):
                b0 = base + par * CHUNK
                in_wait(par)
                if p == 0:
                    @pl.when(j >= 1)
                    def _():
                        out_wait(par)
                else:
                    out_wait(par)
                gather_chunk(b0, par)
                out_copy(c, b0, par)
                nxt = b0 + 2 * CHUNK
                if p == 0:
                    nc = jnp.where(nxt >= BATCH, c1, c)
                    in_copy(nc, pl.multiple_of(lax.rem(nxt, BATCH), CHUNK), par)
                else:
                    @pl.when(nxt < BATCH)
                    def _():
                        in_copy(c, nxt, par)
            return carry

        lax.fori_loop(0, NCH // 2, pair, 0)
        if p == 0:
            h_row = pltpu.async_copy(knobs_t.at[c1], row_v, sem_row)
    out_wait(0)
    out_wait(1)


def kernel(in_repr, mask_idx, prototype_knobs):
    out_t = _proto_add_t(in_repr.T, mask_idx.astype(jnp.int32), prototype_knobs.T)
    return out_t.T


# speedup vs baseline: 2.7025x; 1.0088x over previous
import functools

import jax
import jax.numpy as jnp
from jax import lax
from jax.experimental import pallas as pl
from jax.experimental.pallas import tpu as pltpu
from jax.experimental.pallas import tpu_sc as plsc

D = 64
BATCH = 16384
NROWS = 100000
NC, NS, L = 2, 16, 16
NW = NC * NS
PASSES = D // NW
CHUNK = 2048
NCH = BATCH // CHUNK
UNROLL = 4

_mesh = plsc.VectorSubcoreMesh(core_axis_name="c", subcore_axis_name="s")


@functools.partial(
    pl.kernel,
    mesh=_mesh,
    out_type=jax.ShapeDtypeStruct((D, BATCH), jnp.float32),
    scratch_types=[
        pltpu.VMEM((NROWS,), jnp.float32),
        pltpu.VMEM((BATCH,), jnp.int32),
        pltpu.VMEM((CHUNK,), jnp.float32),
        pltpu.VMEM((CHUNK,), jnp.float32),
        pltpu.VMEM((CHUNK,), jnp.float32),
        pltpu.VMEM((CHUNK,), jnp.float32),
        pltpu.SemaphoreType.DMA,
        pltpu.SemaphoreType.DMA,
        pltpu.SemaphoreType.DMA,
        pltpu.SemaphoreType.DMA,
        pltpu.SemaphoreType.DMA,
        pltpu.SemaphoreType.DMA,
    ],
    compiler_params=pltpu.CompilerParams(
        use_tc_tiling_on_sc=True,
        needs_layout_passes=False,
        disable_bounds_checks=True,
        disable_semaphore_checks=True,
        skip_device_barrier=True,
    ),
)
def _proto_add_t(in_t, idx_hbm, knobs_t, out_t,
                 row_v, idx_v, ib0, ib1, ob0, ob1,
                 sem_row, sem_idx, sem_i0, sem_i1, sem_o0, sem_o1):
    w = lax.axis_index("s") * NC + lax.axis_index("c")
    ib = (ib0, ib1)
    ob = (ob0, ob1)
    sem_i = (sem_i0, sem_i1)
    sem_o = (sem_o0, sem_o1)
    c0 = w
    c1 = NW + w

    def in_copy(c, b0, par):
        return pltpu.async_copy(
            in_t.at[c].at[pl.ds(b0, CHUNK)], ib[par], sem_i[par]
        )

    def out_copy(c, b0, par):
        return pltpu.async_copy(
            ob[par], out_t.at[c].at[pl.ds(b0, CHUNK)], sem_o[par]
        )

    def in_wait(par):
        pltpu.make_async_copy(
            in_t.at[c0].at[pl.ds(0, CHUNK)], ib[par], sem_i[par]
        ).wait()

    def out_wait(par):
        pltpu.make_async_copy(
            ob[par], out_t.at[c0].at[pl.ds(0, CHUNK)], sem_o[par]
        ).wait()

    def row_wait():
        pltpu.make_async_copy(knobs_t.at[c0], row_v, sem_row).wait()

    def gather_chunk(b0, par):
        @plsc.parallel_loop(0, CHUNK, L, unroll=UNROLL)
        def _gather_add(k, ibuf=ib[par], obuf=ob[par]):
            s = pl.ds(k, L)
            i16 = idx_v[pl.ds(b0 + k, L)]
            g = plsc.load_gather(row_v, [i16])
            obuf[s] = ibuf[s] + g

    h_idx = pltpu.async_copy(idx_hbm, idx_v, sem_idx)
    in_copy(c0, 0, 0)
    in_copy(c0, CHUNK, 1)
    pltpu.async_copy(knobs_t.at[c0], row_v, sem_row)
    h_idx.wait()

    NPAIR = PASSES * NCH // 2

    def pair(jj, carry):
        @pl.when(lax.rem(jj, NCH // 2) == 0)
        def _():
            row_wait()
        c = jnp.where(jj < NCH // 2, c0, c1)
        base = pl.multiple_of(lax.rem(jj, NCH // 2) * (2 * CHUNK), 2 * CHUNK)
        for par in range(2):
            b0 = base + par * CHUNK
            in_wait(par)
            @pl.when(jj >= 1)
            def _():
                out_wait(par)
            gather_chunk(b0, par)
            if par == 1:
                @pl.when(jj == NCH // 2 - 1)
                def _():
                    pltpu.async_copy(knobs_t.at[c1], row_v, sem_row)
            out_copy(c, b0, par)
            g = 2 * jj + par + 2
            @pl.when(g < 2 * NPAIR)
            def _():
                nc = jnp.where(g >= NCH, c1, c0)
                in_copy(nc, pl.multiple_of(lax.rem(g * CHUNK, BATCH), CHUNK), par)
        return carry

    lax.fori_loop(0, NPAIR, pair, 0)
    out_wait(0)
    out_wait(1)


def kernel(in_repr, mask_idx, prototype_knobs):
    out_t = _proto_add_t(in_repr.T, mask_idx.astype(jnp.int32), prototype_knobs.T)
    return out_t.T


# speedup vs baseline: 2.7101x; 1.0028x over previous
import functools

import jax
import jax.numpy as jnp
from jax import lax
from jax.experimental import pallas as pl
from jax.experimental.pallas import tpu as pltpu
from jax.experimental.pallas import tpu_sc as plsc

D = 64
BATCH = 16384
NROWS = 100000
NC, NS, L = 2, 16, 16
NW = NC * NS
PASSES = D // NW
CHUNK = 2048
NCH = BATCH // CHUNK
UNROLL = 8

_mesh = plsc.VectorSubcoreMesh(core_axis_name="c", subcore_axis_name="s")


@functools.partial(
    pl.kernel,
    mesh=_mesh,
    out_type=jax.ShapeDtypeStruct((D, BATCH), jnp.float32),
    scratch_types=[
        pltpu.VMEM((NROWS,), jnp.float32),
        pltpu.VMEM((BATCH,), jnp.int32),
        pltpu.VMEM((CHUNK,), jnp.float32),
        pltpu.VMEM((CHUNK,), jnp.float32),
        pltpu.VMEM((CHUNK,), jnp.float32),
        pltpu.VMEM((CHUNK,), jnp.float32),
        pltpu.SemaphoreType.DMA,
        pltpu.SemaphoreType.DMA,
        pltpu.SemaphoreType.DMA,
        pltpu.SemaphoreType.DMA,
        pltpu.SemaphoreType.DMA,
        pltpu.SemaphoreType.DMA,
    ],
    compiler_params=pltpu.CompilerParams(
        use_tc_tiling_on_sc=True,
        needs_layout_passes=False,
        disable_bounds_checks=True,
        disable_semaphore_checks=True,
        skip_device_barrier=True,
    ),
)
def _proto_add_t(in_t, idx_hbm, knobs_t, out_t,
                 row_v, idx_v, ib0, ib1, ob0, ob1,
                 sem_row, sem_idx, sem_i0, sem_i1, sem_o0, sem_o1):
    w = lax.axis_index("s") * NC + lax.axis_index("c")
    ib = (ib0, ib1)
    ob = (ob0, ob1)
    sem_i = (sem_i0, sem_i1)
    sem_o = (sem_o0, sem_o1)
    c0 = w
    c1 = NW + w

    def in_copy(c, b0, par):
        return pltpu.async_copy(
            in_t.at[c].at[pl.ds(b0, CHUNK)], ib[par], sem_i[par]
        )

    def out_copy(c, b0, par):
        return pltpu.async_copy(
            ob[par], out_t.at[c].at[pl.ds(b0, CHUNK)], sem_o[par]
        )

    def in_wait(par):
        pltpu.make_async_copy(
            in_t.at[c0].at[pl.ds(0, CHUNK)], ib[par], sem_i[par]
        ).wait()

    def out_wait(par):
        pltpu.make_async_copy(
            ob[par], out_t.at[c0].at[pl.ds(0, CHUNK)], sem_o[par]
        ).wait()

    def row_wait():
        pltpu.make_async_copy(knobs_t.at[c0], row_v, sem_row).wait()

    def gather_chunk(b0, par):
        @plsc.parallel_loop(0, CHUNK, L, unroll=UNROLL)
        def _gather_add(k, ibuf=ib[par], obuf=ob[par]):
            s = pl.ds(k, L)
            i16 = idx_v[pl.ds(b0 + k, L)]
            g = plsc.load_gather(row_v, [i16])
            obuf[s] = ibuf[s] + g

    h_idx = pltpu.async_copy(idx_hbm, idx_v, sem_idx)
    in_copy(c0, 0, 0)
    in_copy(c0, CHUNK, 1)
    pltpu.async_copy(knobs_t.at[c0], row_v, sem_row)
    h_idx.wait()

    NPAIR = PASSES * NCH // 2

    def pair(jj, carry):
        @pl.when(lax.rem(jj, NCH // 2) == 0)
        def _():
            row_wait()
        c = jnp.where(jj < NCH // 2, c0, c1)
        base = pl.multiple_of(lax.rem(jj, NCH // 2) * (2 * CHUNK), 2 * CHUNK)
        for par in range(2):
            b0 = base + par * CHUNK
            in_wait(par)
            @pl.when(jj >= 1)
            def _():
                out_wait(par)
            gather_chunk(b0, par)
            if par == 1:
                @pl.when(jj == NCH // 2 - 1)
                def _():
                    pltpu.async_copy(knobs_t.at[c1], row_v, sem_row)
            out_copy(c, b0, par)
            g = 2 * jj + par + 2
            @pl.when(g < 2 * NPAIR)
            def _():
                nc = jnp.where(g >= NCH, c1, c0)
                in_copy(nc, pl.multiple_of(lax.rem(g * CHUNK, BATCH), CHUNK), par)
        return carry

    lax.fori_loop(0, NPAIR, pair, 0)
    out_wait(0)
    out_wait(1)


def kernel(in_repr, mask_idx, prototype_knobs):
    out_t = _proto_add_t(in_repr.T, mask_idx.astype(jnp.int32), prototype_knobs.T)
    return out_t.T
